# Initial kernel scaffold; baseline (speedup 1.0000x reference)
import jax
import jax.numpy as jnp
from jax.experimental import pallas as pl


def kernel(x, token_type_ids, W1a, b1a, W1b, b1b, W2a, b2a, W2b, b2b, type_table, pos_table, gamma, beta):
    raise NotImplementedError("write your pallas kernel here")


# speedup vs baseline: 3.1755x; 3.1755x over previous
import jax
import jax.numpy as jnp
from jax.experimental import pallas as pl
from jax.experimental.pallas import tpu as pltpu

B, L, D1, D2, DM, MAXLEN = 4, 2048, 512, 1024, 2048, 4096
BLK = 128
NTOK = B * L
NBLK = NTOK // BLK
LBLK = L // BLK
EPS = 1e-5


def _gelu(v):
    return 0.5 * v * (1.0 + jax.lax.erf(v * (2.0 ** -0.5)))


def _body(x_ref, m_ref, w1a, b1a, w1b, b1b, w2a, b2a, w2b, b2b,
          ttab, pos, gamma, beta, out_ref):
    x = x_ref[...]
    h1 = jnp.dot(_gelu(jnp.dot(x[:, :D1], w1a[...],
                               preferred_element_type=jnp.float32) + b1a[...]),
                 w1b[...], preferred_element_type=jnp.float32) + b1b[...]
    h2 = jnp.dot(_gelu(jnp.dot(x, w2a[...],
                               preferred_element_type=jnp.float32) + b2a[...]),
                 w2b[...], preferred_element_type=jnp.float32) + b2b[...]
    m = m_ref[...]
    o = h1 * m + h2 * (1.0 - m)
    o = o + ttab[0:1, :] * m + ttab[1:2, :] * (1.0 - m)
    o = o + pos[...]
    mu = jnp.mean(o, axis=-1, keepdims=True)
    c = o - mu
    var = jnp.mean(c * c, axis=-1, keepdims=True)
    out_ref[...] = c * jax.lax.rsqrt(var + EPS) * gamma[...] + beta[...]


def kernel(x, token_type_ids, W1a, b1a, W1b, b1b, W2a, b2a, W2b, b2b,
           type_table, pos_table, gamma, beta):
    xf = x.reshape(NTOK, D2)
    m0 = (token_type_ids.reshape(NTOK, 1) == 0).astype(jnp.float32)
    full = lambda s: pl.BlockSpec(s, lambda i: (0,) * len(s))
    out = pl.pallas_call(
        _body,
        grid=(NBLK,),
        in_specs=[
            pl.BlockSpec((BLK, D2), lambda i: (i, 0)),
            pl.BlockSpec((BLK, 1), lambda i: (i, 0)),
            full((D1, DM)), full((1, DM)),
            full((DM, DM)), full((1, DM)),
            full((D2, DM)), full((1, DM)),
            full((DM, DM)), full((1, DM)),
            full((2, DM)),
            pl.BlockSpec((BLK, DM), lambda i: (i % LBLK, 0)),
            full((1, DM)), full((1, DM)),
        ],
        out_specs=pl.BlockSpec((BLK, DM), lambda i: (i, 0)),
        out_shape=jax.ShapeDtypeStruct((NTOK, DM), jnp.float32),
        compiler_params=pltpu.CompilerParams(
            dimension_semantics=("arbitrary",)),
    )(xf, m0, W1a, b1a.reshape(1, DM), W1b, b1b.reshape(1, DM),
      W2a, b2a.reshape(1, DM), W2b, b2b.reshape(1, DM),
      type_table, pos_table, gamma.reshape(1, DM), beta.reshape(1, DM))
    return out.reshape(B, L, DM)
